# baseline (speedup 1.0000x reference)
import jax
import jax.numpy as jnp
from jax.experimental import pallas as pl
from jax.experimental.pallas import tpu as pltpu

_H1, _W1, _C0, _C1 = 128, 128, 3, 16
_H2, _W2, _C2 = 64, 64, 32
_P1 = _H1 * _W1
_P2 = _H2 * _W2
_P3 = (_H2 // 2) * (_W2 // 2)


def _im2col_dot(xpad, w, *, Cin, W, P):
    ishape = (Cin, P) if Cin % 8 == 0 else (1, P)
    col = jax.lax.broadcasted_iota(jnp.int32, ishape, 1) & (W - 1)
    mask_l = col == 0
    mask_r = col == (W - 1)
    taps = []
    for kh in range(3):
        for kw in range(3):
            start = 2 * W + (kh - 1) * W + (kw - 1)
            s = xpad[:, start:start + P]
            if kw == 0:
                s = jnp.where(mask_l, 0.0, s)
            elif kw == 2:
                s = jnp.where(mask_r, 0.0, s)
            taps.append(s)
    patch = jnp.concatenate(taps, axis=0).astype(jnp.bfloat16)
    return jnp.dot(w.astype(jnp.bfloat16), patch,
                   preferred_element_type=jnp.float32)


def _pool_bias_relu(conv, b, scr, *, C, H, W):
    scr[...] = conv.reshape(C, H, W)
    hp = jnp.maximum(scr[:, 0::2, :], scr[:, 1::2, :]
                     ).astype(jnp.bfloat16)
    flat = hp.reshape(C * (H // 2), W)
    row = jax.lax.broadcasted_iota(jnp.int32, (W, W // 2), 0)
    col2 = 2 * jax.lax.broadcasted_iota(jnp.int32, (W, W // 2), 1)
    sel_e = (row == col2).astype(flat.dtype)
    sel_o = (row == col2 + 1).astype(flat.dtype)
    wp = jnp.maximum(
        jnp.dot(flat, sel_e, preferred_element_type=jnp.float32),
        jnp.dot(flat, sel_o, preferred_element_type=jnp.float32))
    y = jnp.maximum(wp.reshape(C, H // 2, W // 2) + b.reshape(C, 1, 1), 0.0)
    return y.reshape(C, (H // 2) * (W // 2))


def _convs_kernel(x_ref, w1_ref, b1_ref, w2_ref, b2_ref, o_ref,
                  xpad_ref, ypad_ref, c1scr_ref, c2scr_ref):
    xpad_ref[:, :2 * _W1] = jnp.zeros((_C0, 2 * _W1), jnp.float32)
    xpad_ref[:, 2 * _W1 + _P1:] = jnp.zeros((_C0, 2 * _W1), jnp.float32)
    xpad_ref[:, 2 * _W1:2 * _W1 + _P1] = x_ref[0]
    conv1 = _im2col_dot(xpad_ref, w1_ref[...], Cin=_C0, W=_W1, P=_P1)
    y1 = _pool_bias_relu(conv1, b1_ref[...], c1scr_ref, C=_C1, H=_H1, W=_W1)

    ypad_ref[:, :2 * _W2] = jnp.zeros((_C1, 2 * _W2), jnp.bfloat16)
    ypad_ref[:, 2 * _W2 + _P2:] = jnp.zeros((_C1, 2 * _W2), jnp.bfloat16)
    ypad_ref[:, 2 * _W2:2 * _W2 + _P2] = y1.astype(jnp.bfloat16)
    conv2 = _im2col_dot(ypad_ref, w2_ref[...], Cin=_C1, W=_W2, P=_P2)
    o_ref[0] = _pool_bias_relu(conv2, b2_ref[...], c2scr_ref,
                               C=_C2, H=_H2, W=_W2)


def _fused_convs(x_flat, w1, b1, w2, b2):
    B = x_flat.shape[0]
    return pl.pallas_call(
        _convs_kernel,
        out_shape=jax.ShapeDtypeStruct((B, _C2, _P3), jnp.float32),
        grid=(B,),
        in_specs=[
            pl.BlockSpec((1, _C0, _P1), lambda b: (b, 0, 0)),
            pl.BlockSpec((_C1, 9 * _C0), lambda b: (0, 0)),
            pl.BlockSpec((_C1, 1), lambda b: (0, 0)),
            pl.BlockSpec((_C2, 9 * _C1), lambda b: (0, 0)),
            pl.BlockSpec((_C2, 1), lambda b: (0, 0)),
        ],
        out_specs=pl.BlockSpec((1, _C2, _P3), lambda b: (b, 0, 0)),
        scratch_shapes=[
            pltpu.VMEM((_C0, _P1 + 4 * _W1), jnp.float32),
            pltpu.VMEM((_C1, _P2 + 4 * _W2), jnp.bfloat16),
            pltpu.VMEM((_C1, _H1, _W1), jnp.float32),
            pltpu.VMEM((_C2, _H2, _W2), jnp.float32),
        ],
        compiler_params=pltpu.CompilerParams(
            dimension_semantics=("arbitrary",)),
    )(x_flat, w1, b1, w2, b2)


def _fc_kernel(x_ref, w1_ref, b1_ref, w2_ref, b2_ref, w3_ref, b3_ref,
               o_ref, acc_ref):
    k = pl.program_id(0)

    @pl.when(k == 0)
    def _():
        acc_ref[...] = jnp.zeros_like(acc_ref)

    acc_ref[...] += jnp.dot(x_ref[...], w1_ref[...],
                            preferred_element_type=jnp.float32)

    @pl.when(k == pl.num_programs(0) - 1)
    def _():
        h1 = acc_ref[...] + b1_ref[...]
        h2 = jnp.dot(h1, w2_ref[...],
                     preferred_element_type=jnp.float32) + b2_ref[...]
        o_ref[...] = (jnp.dot(h2, w3_ref[...],
                              preferred_element_type=jnp.float32)
                      + b3_ref[...]).astype(o_ref.dtype)


def _fc_head(x_flat, w1, b1, w2, b2, w3, b3, *, tk=8192):
    B, K = x_flat.shape
    H1, H2, NC = w1.shape[1], w2.shape[1], w3.shape[1]
    return pl.pallas_call(
        _fc_kernel,
        out_shape=jax.ShapeDtypeStruct((B, NC), jnp.float32),
        grid=(K // tk,),
        in_specs=[
            pl.BlockSpec((B, tk), lambda k: (0, k)),
            pl.BlockSpec((tk, H1), lambda k: (k, 0)),
            pl.BlockSpec((1, H1), lambda k: (0, 0)),
            pl.BlockSpec((H1, H2), lambda k: (0, 0)),
            pl.BlockSpec((1, H2), lambda k: (0, 0)),
            pl.BlockSpec((H2, NC), lambda k: (0, 0)),
            pl.BlockSpec((1, NC), lambda k: (0, 0)),
        ],
        out_specs=pl.BlockSpec((B, NC), lambda k: (0, 0)),
        scratch_shapes=[pltpu.VMEM((B, H1), jnp.float32)],
        compiler_params=pltpu.CompilerParams(
            dimension_semantics=("arbitrary",)),
    )(x_flat, w1, b1, w2, b2, w3, b3)


@jax.jit
def kernel(x, conv1_w, conv1_b, conv2_w, conv2_b,
           fc1_w, fc1_b, fc2_w, fc2_b, fc3_w, fc3_b):
    B = x.shape[0]
    x_flat = x.astype(jnp.float32).reshape(B, _C0, _P1)
    y2 = _fused_convs(x_flat, conv1_w, conv1_b, conv2_w, conv2_b)
    flat = y2.reshape(B, _C2 * _P3)
    return _fc_head(flat, fc1_w, fc1_b, fc2_w, fc2_b, fc3_w, fc3_b)


# speedup vs baseline: 1.1007x; 1.1007x over previous
import jax
import jax.numpy as jnp
from jax.experimental import pallas as pl
from jax.experimental.pallas import tpu as pltpu

_H1, _W1, _C0, _C1 = 128, 128, 3, 16
_H2, _W2, _C2 = 64, 64, 32
_P1 = _H1 * _W1
_P2 = _H2 * _W2
_P3 = (_H2 // 2) * (_W2 // 2)
_IPB = 2


def _im2col_dot(xpad, i, w, *, W, P):
    col = jax.lax.broadcasted_iota(jnp.int32, (1, P), 1) & (W - 1)
    mask_l = col == 0
    mask_r = col == (W - 1)
    taps = []
    for kh in range(3):
        for kw in range(3):
            start = 2 * W + (kh - 1) * W + (kw - 1)
            s = xpad[i, :, start:start + P]
            if kw == 0:
                s = jnp.where(mask_l, 0.0, s)
            elif kw == 2:
                s = jnp.where(mask_r, 0.0, s)
            taps.append(s)
    patch = jnp.concatenate(taps, axis=0)
    return jnp.dot(w, patch, preferred_element_type=jnp.float32)


def _pool_bias_relu(conv, b, scr, i, *, C, H, W):
    scr[i] = conv.reshape(C, H, W)
    hp = jnp.maximum(scr[i, :, 0::2, :], scr[i, :, 1::2, :])
    flat = hp.reshape(C * (H // 2), W)
    row = jax.lax.broadcasted_iota(jnp.int32, (W, W // 2), 0)
    col2 = 2 * jax.lax.broadcasted_iota(jnp.int32, (W, W // 2), 1)
    sel_e = (row == col2).astype(jnp.float32)
    sel_o = (row == col2 + 1).astype(jnp.float32)
    wp = jnp.maximum(
        jnp.dot(flat, sel_e, preferred_element_type=jnp.float32),
        jnp.dot(flat, sel_o, preferred_element_type=jnp.float32))
    y = jnp.maximum(wp.reshape(C, H // 2, W // 2) + b.reshape(C, 1, 1), 0.0)
    return y.reshape(C, (H // 2) * (W // 2))


def _convs_kernel(x_ref, w1_ref, b1_ref, w2_ref, b2_ref, o_ref,
                  xpad_ref, ypad_ref, c1scr_ref, c2scr_ref):
    w1 = w1_ref[...]
    b1 = b1_ref[...]
    w2 = w2_ref[...]
    b2 = b2_ref[...]
    for i in range(_IPB):
        xpad_ref[i, :, :2 * _W1] = jnp.zeros((_C0, 2 * _W1), jnp.float32)
        xpad_ref[i, :, 2 * _W1 + _P1:] = jnp.zeros((_C0, 2 * _W1),
                                                   jnp.float32)
        xpad_ref[i, :, 2 * _W1:2 * _W1 + _P1] = x_ref[i]
        conv1 = _im2col_dot(xpad_ref, i, w1, W=_W1, P=_P1)
        y1 = _pool_bias_relu(conv1, b1, c1scr_ref, i, C=_C1, H=_H1, W=_W1)

        ypad_ref[i, :, :2 * _W2] = jnp.zeros((_C1, 2 * _W2), jnp.float32)
        ypad_ref[i, :, 2 * _W2 + _P2:] = jnp.zeros((_C1, 2 * _W2),
                                                   jnp.float32)
        ypad_ref[i, :, 2 * _W2:2 * _W2 + _P2] = y1
        conv2 = _im2col_dot(ypad_ref, i, w2, W=_W2, P=_P2)
        o_ref[i] = _pool_bias_relu(conv2, b2, c2scr_ref, i,
                                   C=_C2, H=_H2, W=_W2)


def _fused_convs(x_flat, w1, b1, w2, b2):
    B = x_flat.shape[0]
    return pl.pallas_call(
        _convs_kernel,
        out_shape=jax.ShapeDtypeStruct((B, _C2, _P3), jnp.float32),
        grid=(B // _IPB,),
        in_specs=[
            pl.BlockSpec((_IPB, _C0, _P1), lambda b: (b, 0, 0)),
            pl.BlockSpec((_C1, 9 * _C0), lambda b: (0, 0)),
            pl.BlockSpec((_C1, 1), lambda b: (0, 0)),
            pl.BlockSpec((_C2, 9 * _C1), lambda b: (0, 0)),
            pl.BlockSpec((_C2, 1), lambda b: (0, 0)),
        ],
        out_specs=pl.BlockSpec((_IPB, _C2, _P3), lambda b: (b, 0, 0)),
        scratch_shapes=[
            pltpu.VMEM((_IPB, _C0, _P1 + 4 * _W1), jnp.float32),
            pltpu.VMEM((_IPB, _C1, _P2 + 4 * _W2), jnp.float32),
            pltpu.VMEM((_IPB, _C1, _H1, _W1), jnp.float32),
            pltpu.VMEM((_IPB, _C2, _H2, _W2), jnp.float32),
        ],
        compiler_params=pltpu.CompilerParams(
            dimension_semantics=("arbitrary",)),
    )(x_flat, w1, b1, w2, b2)


def _fc_kernel(x_ref, w1_ref, b1_ref, w2_ref, b2_ref, w3_ref, b3_ref,
               o_ref, acc_ref):
    k = pl.program_id(0)

    @pl.when(k == 0)
    def _():
        acc_ref[...] = jnp.zeros_like(acc_ref)

    acc_ref[...] += jnp.dot(x_ref[...], w1_ref[...],
                            preferred_element_type=jnp.float32)

    @pl.when(k == pl.num_programs(0) - 1)
    def _():
        h1 = acc_ref[...] + b1_ref[...]
        h2 = jnp.dot(h1, w2_ref[...],
                     preferred_element_type=jnp.float32) + b2_ref[...]
        o_ref[...] = (jnp.dot(h2, w3_ref[...],
                              preferred_element_type=jnp.float32)
                      + b3_ref[...]).astype(o_ref.dtype)


def _fc_head(x_flat, w1, b1, w2, b2, w3, b3, *, tk=4096):
    B, K = x_flat.shape
    H1, H2, NC = w1.shape[1], w2.shape[1], w3.shape[1]
    return pl.pallas_call(
        _fc_kernel,
        out_shape=jax.ShapeDtypeStruct((B, NC), jnp.float32),
        grid=(K // tk,),
        in_specs=[
            pl.BlockSpec((B, tk), lambda k: (0, k)),
            pl.BlockSpec((tk, H1), lambda k: (k, 0)),
            pl.BlockSpec((1, H1), lambda k: (0, 0)),
            pl.BlockSpec((H1, H2), lambda k: (0, 0)),
            pl.BlockSpec((1, H2), lambda k: (0, 0)),
            pl.BlockSpec((H2, NC), lambda k: (0, 0)),
            pl.BlockSpec((1, NC), lambda k: (0, 0)),
        ],
        out_specs=pl.BlockSpec((B, NC), lambda k: (0, 0)),
        scratch_shapes=[pltpu.VMEM((B, H1), jnp.float32)],
        compiler_params=pltpu.CompilerParams(
            dimension_semantics=("arbitrary",)),
    )(x_flat, w1, b1, w2, b2, w3, b3)


@jax.jit
def kernel(x, conv1_w, conv1_b, conv2_w, conv2_b,
           fc1_w, fc1_b, fc2_w, fc2_b, fc3_w, fc3_b):
    B = x.shape[0]
    x_flat = x.astype(jnp.float32).reshape(B, _C0, _P1)
    y2 = _fused_convs(x_flat, conv1_w, conv1_b, conv2_w, conv2_b)
    flat = y2.reshape(B, _C2 * _P3)
    return _fc_head(flat, fc1_w, fc1_b, fc2_w, fc2_b, fc3_w, fc3_b)


# speedup vs baseline: 1.1203x; 1.0178x over previous
import jax
import jax.numpy as jnp
from jax.experimental import pallas as pl
from jax.experimental.pallas import tpu as pltpu

_H1, _W1, _C0, _C1 = 128, 128, 3, 16
_H2, _W2, _C2 = 64, 64, 32
_P1 = _H1 * _W1
_P2 = _H2 * _W2
_P3 = (_H2 // 2) * (_W2 // 2)
_IPB = 4


def _im2col_dot(xpad, i, w, *, W, P):
    col = jax.lax.broadcasted_iota(jnp.int32, (1, P), 1) & (W - 1)
    mask_l = col == 0
    mask_r = col == (W - 1)
    taps = []
    for kh in range(3):
        for kw in range(3):
            start = 2 * W + (kh - 1) * W + (kw - 1)
            s = xpad[i, :, start:start + P]
            if kw == 0:
                s = jnp.where(mask_l, 0.0, s)
            elif kw == 2:
                s = jnp.where(mask_r, 0.0, s)
            taps.append(s)
    patch = jnp.concatenate(taps, axis=0)
    return jnp.dot(w, patch, preferred_element_type=jnp.float32)


def _pool_bias_relu(conv, b, scr, i, *, C, H, W):
    scr[i] = conv.reshape(C, H, W)
    hp = jnp.maximum(scr[i, :, 0::2, :], scr[i, :, 1::2, :])
    flat = hp.reshape(C * (H // 2), W)
    row = jax.lax.broadcasted_iota(jnp.int32, (W, W // 2), 0)
    col2 = 2 * jax.lax.broadcasted_iota(jnp.int32, (W, W // 2), 1)
    sel_e = (row == col2).astype(jnp.float32)
    sel_o = (row == col2 + 1).astype(jnp.float32)
    wp = jnp.maximum(
        jnp.dot(flat, sel_e, preferred_element_type=jnp.float32),
        jnp.dot(flat, sel_o, preferred_element_type=jnp.float32))
    y = jnp.maximum(wp.reshape(C, H // 2, W // 2) + b.reshape(C, 1, 1), 0.0)
    return y.reshape(C, (H // 2) * (W // 2))


def _convs_kernel(x_ref, w1_ref, b1_ref, w2_ref, b2_ref, o_ref,
                  xpad_ref, ypad_ref, c1scr_ref, c2scr_ref):
    w1 = w1_ref[...]
    b1 = b1_ref[...]
    w2 = w2_ref[...]
    b2 = b2_ref[...]
    for i in range(_IPB):
        xpad_ref[i, :, :2 * _W1] = jnp.zeros((_C0, 2 * _W1), jnp.float32)
        xpad_ref[i, :, 2 * _W1 + _P1:] = jnp.zeros((_C0, 2 * _W1),
                                                   jnp.float32)
        xpad_ref[i, :, 2 * _W1:2 * _W1 + _P1] = x_ref[i]
        conv1 = _im2col_dot(xpad_ref, i, w1, W=_W1, P=_P1)
        y1 = _pool_bias_relu(conv1, b1, c1scr_ref, i, C=_C1, H=_H1, W=_W1)

        ypad_ref[i, :, :2 * _W2] = jnp.zeros((_C1, 2 * _W2), jnp.float32)
        ypad_ref[i, :, 2 * _W2 + _P2:] = jnp.zeros((_C1, 2 * _W2),
                                                   jnp.float32)
        ypad_ref[i, :, 2 * _W2:2 * _W2 + _P2] = y1
        conv2 = _im2col_dot(ypad_ref, i, w2, W=_W2, P=_P2)
        o_ref[i] = _pool_bias_relu(conv2, b2, c2scr_ref, i,
                                   C=_C2, H=_H2, W=_W2)


def _fused_convs(x_flat, w1, b1, w2, b2):
    B = x_flat.shape[0]
    return pl.pallas_call(
        _convs_kernel,
        out_shape=jax.ShapeDtypeStruct((B, _C2, _P3), jnp.float32),
        grid=(B // _IPB,),
        in_specs=[
            pl.BlockSpec((_IPB, _C0, _P1), lambda b: (b, 0, 0)),
            pl.BlockSpec((_C1, 9 * _C0), lambda b: (0, 0)),
            pl.BlockSpec((_C1, 1), lambda b: (0, 0)),
            pl.BlockSpec((_C2, 9 * _C1), lambda b: (0, 0)),
            pl.BlockSpec((_C2, 1), lambda b: (0, 0)),
        ],
        out_specs=pl.BlockSpec((_IPB, _C2, _P3), lambda b: (b, 0, 0)),
        scratch_shapes=[
            pltpu.VMEM((_IPB, _C0, _P1 + 4 * _W1), jnp.float32),
            pltpu.VMEM((_IPB, _C1, _P2 + 4 * _W2), jnp.float32),
            pltpu.VMEM((_IPB, _C1, _H1, _W1), jnp.float32),
            pltpu.VMEM((_IPB, _C2, _H2, _W2), jnp.float32),
        ],
        compiler_params=pltpu.CompilerParams(
            dimension_semantics=("arbitrary",)),
    )(x_flat, w1, b1, w2, b2)


def _fc_kernel(x_ref, w1_ref, b1_ref, w2_ref, b2_ref, w3_ref, b3_ref,
               o_ref, acc_ref):
    k = pl.program_id(0)

    @pl.when(k == 0)
    def _():
        acc_ref[...] = jnp.zeros_like(acc_ref)

    acc_ref[...] += jnp.dot(x_ref[...], w1_ref[...],
                            preferred_element_type=jnp.float32)

    @pl.when(k == pl.num_programs(0) - 1)
    def _():
        h1 = acc_ref[...] + b1_ref[...]
        h2 = jnp.dot(h1, w2_ref[...],
                     preferred_element_type=jnp.float32) + b2_ref[...]
        o_ref[...] = (jnp.dot(h2, w3_ref[...],
                              preferred_element_type=jnp.float32)
                      + b3_ref[...]).astype(o_ref.dtype)


def _fc_head(x_flat, w1, b1, w2, b2, w3, b3, *, tk=4096):
    B, K = x_flat.shape
    H1, H2, NC = w1.shape[1], w2.shape[1], w3.shape[1]
    return pl.pallas_call(
        _fc_kernel,
        out_shape=jax.ShapeDtypeStruct((B, NC), jnp.float32),
        grid=(K // tk,),
        in_specs=[
            pl.BlockSpec((B, tk), lambda k: (0, k)),
            pl.BlockSpec((tk, H1), lambda k: (k, 0)),
            pl.BlockSpec((1, H1), lambda k: (0, 0)),
            pl.BlockSpec((H1, H2), lambda k: (0, 0)),
            pl.BlockSpec((1, H2), lambda k: (0, 0)),
            pl.BlockSpec((H2, NC), lambda k: (0, 0)),
            pl.BlockSpec((1, NC), lambda k: (0, 0)),
        ],
        out_specs=pl.BlockSpec((B, NC), lambda k: (0, 0)),
        scratch_shapes=[pltpu.VMEM((B, H1), jnp.float32)],
        compiler_params=pltpu.CompilerParams(
            dimension_semantics=("arbitrary",)),
    )(x_flat, w1, b1, w2, b2, w3, b3)


@jax.jit
def kernel(x, conv1_w, conv1_b, conv2_w, conv2_b,
           fc1_w, fc1_b, fc2_w, fc2_b, fc3_w, fc3_b):
    B = x.shape[0]
    x_flat = x.astype(jnp.float32).reshape(B, _C0, _P1)
    y2 = _fused_convs(x_flat, conv1_w, conv1_b, conv2_w, conv2_b)
    flat = y2.reshape(B, _C2 * _P3)
    return _fc_head(flat, fc1_w, fc1_b, fc2_w, fc2_b, fc3_w, fc3_b)


# speedup vs baseline: 1.2433x; 1.1098x over previous
import jax
import jax.numpy as jnp
from jax.experimental import pallas as pl
from jax.experimental.pallas import tpu as pltpu

_H1, _W1, _C0, _C1 = 128, 128, 3, 16
_H2, _W2, _C2 = 64, 64, 32
_P1 = _H1 * _W1
_P2 = _H2 * _W2
_P3 = (_H2 // 2) * (_W2 // 2)
_IPB = 4


def _im2col_dot(xpad, i, w, *, W, P):
    col = jax.lax.broadcasted_iota(jnp.int32, (1, P), 1) & (W - 1)
    mask_l = col == 0
    mask_r = col == (W - 1)
    taps = []
    for kh in range(3):
        for kw in range(3):
            start = 2 * W + (kh - 1) * W + (kw - 1)
            s = xpad[i, :, start:start + P]
            if kw == 0:
                s = jnp.where(mask_l, 0.0, s)
            elif kw == 2:
                s = jnp.where(mask_r, 0.0, s)
            taps.append(s)
    patch = jnp.concatenate(taps, axis=0)
    return jnp.dot(w, patch, preferred_element_type=jnp.float32)


def _pool_bias_relu(conv, b, scr, i, *, C, H, W):
    scr[i] = conv.reshape(C, H, W)
    hp = jnp.maximum(scr[i, :, 0::2, :], scr[i, :, 1::2, :])
    flat = hp.reshape(C * (H // 2), W)
    row = jax.lax.broadcasted_iota(jnp.int32, (W, W // 2), 0)
    col2 = 2 * jax.lax.broadcasted_iota(jnp.int32, (W, W // 2), 1)
    sel_e = (row == col2).astype(jnp.float32)
    sel_o = (row == col2 + 1).astype(jnp.float32)
    wp = jnp.maximum(
        jnp.dot(flat, sel_e, preferred_element_type=jnp.float32),
        jnp.dot(flat, sel_o, preferred_element_type=jnp.float32))
    y = jnp.maximum(wp.reshape(C, H // 2, W // 2) + b.reshape(C, 1, 1), 0.0)
    return y


def _net_kernel(x_ref, w1_ref, b1_ref, w2_ref, b2_ref,
                fw1_ref, fb1_ref, fw2_ref, fb2_ref, fw3_ref, fb3_ref,
                o_ref, xpad_ref, ypad_ref, c1scr_ref, c2scr_ref, flat_ref):
    g = pl.program_id(0)
    w1 = w1_ref[...]
    b1 = b1_ref[...]
    w2 = w2_ref[...]
    b2 = b2_ref[...]
    for i in range(_IPB):
        xpad_ref[i, :, :2 * _W1] = jnp.zeros((_C0, 2 * _W1), jnp.float32)
        xpad_ref[i, :, 2 * _W1 + _P1:] = jnp.zeros((_C0, 2 * _W1),
                                                   jnp.float32)
        xpad_ref[i, :, 2 * _W1:2 * _W1 + _P1] = x_ref[i]
        conv1 = _im2col_dot(xpad_ref, i, w1, W=_W1, P=_P1)
        y1 = _pool_bias_relu(conv1, b1, c1scr_ref, i, C=_C1, H=_H1, W=_W1)

        ypad_ref[i, :, :2 * _W2] = jnp.zeros((_C1, 2 * _W2), jnp.float32)
        ypad_ref[i, :, 2 * _W2 + _P2:] = jnp.zeros((_C1, 2 * _W2),
                                                   jnp.float32)
        ypad_ref[i, :, 2 * _W2:2 * _W2 + _P2] = y1.reshape(_C1, _P2)
        conv2 = _im2col_dot(ypad_ref, i, w2, W=_W2, P=_P2)
        y2 = _pool_bias_relu(conv2, b2, c2scr_ref, i, C=_C2, H=_H2, W=_W2)

        flat_ref[pl.ds(g * _IPB + i, 1)] = y2.reshape(1, _C2, _P3)

    @pl.when(g == pl.num_programs(0) - 1)
    def _():
        B = flat_ref.shape[0]
        flat = flat_ref[...].reshape(B, _C2 * _P3)
        h1 = jnp.dot(flat, fw1_ref[...],
                     preferred_element_type=jnp.float32) + fb1_ref[...]
        h2 = jnp.dot(h1, fw2_ref[...],
                     preferred_element_type=jnp.float32) + fb2_ref[...]
        o_ref[...] = (jnp.dot(h2, fw3_ref[...],
                              preferred_element_type=jnp.float32)
                      + fb3_ref[...]).astype(o_ref.dtype)


@jax.jit
def kernel(x, conv1_w, conv1_b, conv2_w, conv2_b,
           fc1_w, fc1_b, fc2_w, fc2_b, fc3_w, fc3_b):
    B = x.shape[0]
    NC = fc3_w.shape[1]
    x_flat = x.astype(jnp.float32).reshape(B, _C0, _P1)
    cparams = pltpu.CompilerParams(dimension_semantics=("arbitrary",))
    return pl.pallas_call(
        _net_kernel,
        out_shape=jax.ShapeDtypeStruct((B, NC), jnp.float32),
        grid=(B // _IPB,),
        in_specs=[
            pl.BlockSpec((_IPB, _C0, _P1), lambda b: (b, 0, 0)),
            pl.BlockSpec((_C1, 9 * _C0), lambda b: (0, 0)),
            pl.BlockSpec((_C1, 1), lambda b: (0, 0)),
            pl.BlockSpec((_C2, 9 * _C1), lambda b: (0, 0)),
            pl.BlockSpec((_C2, 1), lambda b: (0, 0)),
            pl.BlockSpec((_C2 * _P3, 128), lambda b: (0, 0)),
            pl.BlockSpec((1, 128), lambda b: (0, 0)),
            pl.BlockSpec((128, 32), lambda b: (0, 0)),
            pl.BlockSpec((1, 32), lambda b: (0, 0)),
            pl.BlockSpec((32, NC), lambda b: (0, 0)),
            pl.BlockSpec((1, NC), lambda b: (0, 0)),
        ],
        out_specs=pl.BlockSpec((B, NC), lambda b: (0, 0)),
        scratch_shapes=[
            pltpu.VMEM((_IPB, _C0, _P1 + 4 * _W1), jnp.float32),
            pltpu.VMEM((_IPB, _C1, _P2 + 4 * _W2), jnp.float32),
            pltpu.VMEM((_IPB, _C1, _H1, _W1), jnp.float32),
            pltpu.VMEM((_IPB, _C2, _H2, _W2), jnp.float32),
            pltpu.VMEM((B, _C2, _P3), jnp.float32),
        ],
        compiler_params=cparams,
    )(x_flat, conv1_w, conv1_b, conv2_w, conv2_b,
      fc1_w, fc1_b, fc2_w, fc2_b, fc3_w, fc3_b)


# speedup vs baseline: 1.2719x; 1.0231x over previous
import jax
import jax.numpy as jnp
from jax.experimental import pallas as pl
from jax.experimental.pallas import tpu as pltpu

_H1, _W1, _C0, _C1 = 128, 128, 3, 16
_H2, _W2, _C2 = 64, 64, 32
_P1 = _H1 * _W1
_P2 = _H2 * _W2
_P3 = (_H2 // 2) * (_W2 // 2)
_IPB = 8


def _im2col_dot(xpad, i, w, *, W, P):
    col = jax.lax.broadcasted_iota(jnp.int32, (1, P), 1) & (W - 1)
    mask_l = col == 0
    mask_r = col == (W - 1)
    taps = []
    for kh in range(3):
        for kw in range(3):
            start = 2 * W + (kh - 1) * W + (kw - 1)
            s = xpad[i, :, start:start + P]
            if kw == 0:
                s = jnp.where(mask_l, 0.0, s)
            elif kw == 2:
                s = jnp.where(mask_r, 0.0, s)
            taps.append(s)
    patch = jnp.concatenate(taps, axis=0)
    return jnp.dot(w, patch, preferred_element_type=jnp.float32)


def _pool_bias_relu(conv, b, scr, i, *, C, H, W):
    scr[i] = conv.reshape(C, H, W)
    hp = jnp.maximum(scr[i, :, 0::2, :], scr[i, :, 1::2, :])
    flat = hp.reshape(C * (H // 2), W)
    row = jax.lax.broadcasted_iota(jnp.int32, (W, W // 2), 0)
    col2 = 2 * jax.lax.broadcasted_iota(jnp.int32, (W, W // 2), 1)
    sel_e = (row == col2).astype(jnp.float32)
    sel_o = (row == col2 + 1).astype(jnp.float32)
    wp = jnp.maximum(
        jnp.dot(flat, sel_e, preferred_element_type=jnp.float32),
        jnp.dot(flat, sel_o, preferred_element_type=jnp.float32))
    y = jnp.maximum(wp.reshape(C, H // 2, W // 2) + b.reshape(C, 1, 1), 0.0)
    return y


def _net_kernel(x_ref, w1_ref, b1_ref, w2_ref, b2_ref,
                fw1_ref, fb1_ref, fw2_ref, fb2_ref, fw3_ref, fb3_ref,
                o_ref, xpad_ref, ypad_ref, c1scr_ref, c2scr_ref, flat_ref):
    g = pl.program_id(0)
    w1 = w1_ref[...]
    b1 = b1_ref[...]
    w2 = w2_ref[...]
    b2 = b2_ref[...]
    for i in range(_IPB):
        xpad_ref[i, :, :2 * _W1] = jnp.zeros((_C0, 2 * _W1), jnp.float32)
        xpad_ref[i, :, 2 * _W1 + _P1:] = jnp.zeros((_C0, 2 * _W1),
                                                   jnp.float32)
        xpad_ref[i, :, 2 * _W1:2 * _W1 + _P1] = x_ref[i]
        conv1 = _im2col_dot(xpad_ref, i, w1, W=_W1, P=_P1)
        y1 = _pool_bias_relu(conv1, b1, c1scr_ref, i, C=_C1, H=_H1, W=_W1)

        ypad_ref[i, :, :2 * _W2] = jnp.zeros((_C1, 2 * _W2), jnp.float32)
        ypad_ref[i, :, 2 * _W2 + _P2:] = jnp.zeros((_C1, 2 * _W2),
                                                   jnp.float32)
        ypad_ref[i, :, 2 * _W2:2 * _W2 + _P2] = y1.reshape(_C1, _P2)
        conv2 = _im2col_dot(ypad_ref, i, w2, W=_W2, P=_P2)
        y2 = _pool_bias_relu(conv2, b2, c2scr_ref, i, C=_C2, H=_H2, W=_W2)

        flat_ref[pl.ds(g * _IPB + i, 1)] = y2.reshape(1, _C2, _P3)

    @pl.when(g == pl.num_programs(0) - 1)
    def _():
        B = flat_ref.shape[0]
        flat = flat_ref[...].reshape(B, _C2 * _P3)
        h1 = jnp.dot(flat, fw1_ref[...],
                     preferred_element_type=jnp.float32) + fb1_ref[...]
        h2 = jnp.dot(h1, fw2_ref[...],
                     preferred_element_type=jnp.float32) + fb2_ref[...]
        o_ref[...] = (jnp.dot(h2, fw3_ref[...],
                              preferred_element_type=jnp.float32)
                      + fb3_ref[...]).astype(o_ref.dtype)


@jax.jit
def kernel(x, conv1_w, conv1_b, conv2_w, conv2_b,
           fc1_w, fc1_b, fc2_w, fc2_b, fc3_w, fc3_b):
    B = x.shape[0]
    NC = fc3_w.shape[1]
    x_flat = x.astype(jnp.float32).reshape(B, _C0, _P1)
    cparams = pltpu.CompilerParams(dimension_semantics=("arbitrary",))
    return pl.pallas_call(
        _net_kernel,
        out_shape=jax.ShapeDtypeStruct((B, NC), jnp.float32),
        grid=(B // _IPB,),
        in_specs=[
            pl.BlockSpec((_IPB, _C0, _P1), lambda b: (b, 0, 0)),
            pl.BlockSpec((_C1, 9 * _C0), lambda b: (0, 0)),
            pl.BlockSpec((_C1, 1), lambda b: (0, 0)),
            pl.BlockSpec((_C2, 9 * _C1), lambda b: (0, 0)),
            pl.BlockSpec((_C2, 1), lambda b: (0, 0)),
            pl.BlockSpec((_C2 * _P3, 128), lambda b: (0, 0)),
            pl.BlockSpec((1, 128), lambda b: (0, 0)),
            pl.BlockSpec((128, 32), lambda b: (0, 0)),
            pl.BlockSpec((1, 32), lambda b: (0, 0)),
            pl.BlockSpec((32, NC), lambda b: (0, 0)),
            pl.BlockSpec((1, NC), lambda b: (0, 0)),
        ],
        out_specs=pl.BlockSpec((B, NC), lambda b: (0, 0)),
        scratch_shapes=[
            pltpu.VMEM((_IPB, _C0, _P1 + 4 * _W1), jnp.float32),
            pltpu.VMEM((_IPB, _C1, _P2 + 4 * _W2), jnp.float32),
            pltpu.VMEM((_IPB, _C1, _H1, _W1), jnp.float32),
            pltpu.VMEM((_IPB, _C2, _H2, _W2), jnp.float32),
            pltpu.VMEM((B, _C2, _P3), jnp.float32),
        ],
        compiler_params=cparams,
    )(x_flat, conv1_w, conv1_b, conv2_w, conv2_b,
      fc1_w, fc1_b, fc2_w, fc2_b, fc3_w, fc3_b)


# speedup vs baseline: 1.3609x; 1.0699x over previous
import jax
import jax.numpy as jnp
from jax.experimental import pallas as pl
from jax.experimental.pallas import tpu as pltpu

_H1, _W1, _C0, _C1 = 128, 128, 3, 16
_H2, _W2, _C2 = 64, 64, 32
_P1 = _H1 * _W1
_P2 = _H2 * _W2
_P3 = (_H2 // 2) * (_W2 // 2)
_IPB = 8


def _edge_masks(W, P):
    col = jax.lax.broadcasted_iota(jnp.int32, (1, P), 1) & (W - 1)
    return col == 0, col == (W - 1)


def _sel_mats(W):
    row = jax.lax.broadcasted_iota(jnp.int32, (W, W // 2), 0)
    col2 = 2 * jax.lax.broadcasted_iota(jnp.int32, (W, W // 2), 1)
    sel_e = (row == col2).astype(jnp.float32)
    sel_o = (row == col2 + 1).astype(jnp.float32)
    return sel_e, sel_o


def _im2col_dot(xpad, i, w, masks, *, W, P):
    mask_l, mask_r = masks
    taps = []
    for kh in range(3):
        for kw in range(3):
            start = 2 * W + (kh - 1) * W + (kw - 1)
            s = xpad[i, :, start:start + P]
            if kw == 0:
                s = jnp.where(mask_l, 0.0, s)
            elif kw == 2:
                s = jnp.where(mask_r, 0.0, s)
            taps.append(s)
    patch = jnp.concatenate(taps, axis=0)
    return jnp.dot(w, patch, preferred_element_type=jnp.float32)


def _pool_bias_relu(conv, b, scr, i, sels, *, C, H, W):
    scr[i] = conv.reshape(C, H, W)
    hp = jnp.maximum(scr[i, :, 0::2, :], scr[i, :, 1::2, :])
    flat = hp.reshape(C * (H // 2), W)
    sel_e, sel_o = sels
    wp = jnp.maximum(
        jnp.dot(flat, sel_e, preferred_element_type=jnp.float32),
        jnp.dot(flat, sel_o, preferred_element_type=jnp.float32))
    y = jnp.maximum(wp.reshape(C, H // 2, W // 2) + b.reshape(C, 1, 1), 0.0)
    return y


def _net_kernel(x_ref, w1_ref, b1_ref, w2_ref, b2_ref,
                fw1_ref, fb1_ref, fw2_ref, fb2_ref, fw3_ref, fb3_ref,
                o_ref, xpad_ref, ypad_ref, c1scr_ref, c2scr_ref, flat_ref):
    g = pl.program_id(0)
    w1 = w1_ref[...]
    b1 = b1_ref[...]
    w2 = w2_ref[...]
    b2 = b2_ref[...]
    masks1 = _edge_masks(_W1, _P1)
    masks2 = _edge_masks(_W2, _P2)
    sels1 = _sel_mats(_W1)
    sels2 = _sel_mats(_W2)

    @pl.when(g == 0)
    def _():
        for i in range(_IPB):
            xpad_ref[i, :, :2 * _W1] = jnp.zeros((_C0, 2 * _W1), jnp.float32)
            xpad_ref[i, :, 2 * _W1 + _P1:] = jnp.zeros((_C0, 2 * _W1),
                                                       jnp.float32)
            ypad_ref[i, :, :2 * _W2] = jnp.zeros((_C1, 2 * _W2), jnp.float32)
            ypad_ref[i, :, 2 * _W2 + _P2:] = jnp.zeros((_C1, 2 * _W2),
                                                       jnp.float32)

    for i in range(_IPB):
        xpad_ref[i, :, 2 * _W1:2 * _W1 + _P1] = x_ref[i]
        conv1 = _im2col_dot(xpad_ref, i, w1, masks1, W=_W1, P=_P1)
        y1 = _pool_bias_relu(conv1, b1, c1scr_ref, i, sels1,
                             C=_C1, H=_H1, W=_W1)

        ypad_ref[i, :, 2 * _W2:2 * _W2 + _P2] = y1.reshape(_C1, _P2)
        conv2 = _im2col_dot(ypad_ref, i, w2, masks2, W=_W2, P=_P2)
        y2 = _pool_bias_relu(conv2, b2, c2scr_ref, i, sels2,
                             C=_C2, H=_H2, W=_W2)

        flat_ref[pl.ds(g * _IPB + i, 1)] = y2.reshape(1, _C2, _P3)

    @pl.when(g == pl.num_programs(0) - 1)
    def _():
        B = flat_ref.shape[0]
        flat = flat_ref[...].reshape(B, _C2 * _P3)
        h1 = jnp.dot(flat, fw1_ref[...],
                     preferred_element_type=jnp.float32) + fb1_ref[...]
        h2 = jnp.dot(h1, fw2_ref[...],
                     preferred_element_type=jnp.float32) + fb2_ref[...]
        o_ref[...] = (jnp.dot(h2, fw3_ref[...],
                              preferred_element_type=jnp.float32)
                      + fb3_ref[...]).astype(o_ref.dtype)


@jax.jit
def kernel(x, conv1_w, conv1_b, conv2_w, conv2_b,
           fc1_w, fc1_b, fc2_w, fc2_b, fc3_w, fc3_b):
    B = x.shape[0]
    NC = fc3_w.shape[1]
    x_flat = x.astype(jnp.float32).reshape(B, _C0, _P1)
    cparams = pltpu.CompilerParams(dimension_semantics=("arbitrary",))
    return pl.pallas_call(
        _net_kernel,
        out_shape=jax.ShapeDtypeStruct((B, NC), jnp.float32),
        grid=(B // _IPB,),
        in_specs=[
            pl.BlockSpec((_IPB, _C0, _P1), lambda b: (b, 0, 0)),
            pl.BlockSpec((_C1, 9 * _C0), lambda b: (0, 0)),
            pl.BlockSpec((_C1, 1), lambda b: (0, 0)),
            pl.BlockSpec((_C2, 9 * _C1), lambda b: (0, 0)),
            pl.BlockSpec((_C2, 1), lambda b: (0, 0)),
            pl.BlockSpec((_C2 * _P3, 128), lambda b: (0, 0)),
            pl.BlockSpec((1, 128), lambda b: (0, 0)),
            pl.BlockSpec((128, 32), lambda b: (0, 0)),
            pl.BlockSpec((1, 32), lambda b: (0, 0)),
            pl.BlockSpec((32, NC), lambda b: (0, 0)),
            pl.BlockSpec((1, NC), lambda b: (0, 0)),
        ],
        out_specs=pl.BlockSpec((B, NC), lambda b: (0, 0)),
        scratch_shapes=[
            pltpu.VMEM((_IPB, _C0, _P1 + 4 * _W1), jnp.float32),
            pltpu.VMEM((_IPB, _C1, _P2 + 4 * _W2), jnp.float32),
            pltpu.VMEM((_IPB, _C1, _H1, _W1), jnp.float32),
            pltpu.VMEM((_IPB, _C2, _H2, _W2), jnp.float32),
            pltpu.VMEM((B, _C2, _P3), jnp.float32),
        ],
        compiler_params=cparams,
    )(x_flat, conv1_w, conv1_b, conv2_w, conv2_b,
      fc1_w, fc1_b, fc2_w, fc2_b, fc3_w, fc3_b)
